# baseline (speedup 1.0000x reference)
import jax
import jax.numpy as jnp
from jax import lax
from jax.experimental import pallas as pl
from jax.experimental.pallas import tpu as pltpu
from jax.experimental.pallas import tpu_sc as plsc

_CUT = 5
_E = 320000
_D = 128

_NC = 2
_NS = 16
_NW = _NC * _NS
_LANES = 16
_MV = _D // _LANES

_PER_W = _E // _NW
_BLK = 400
_NB = _PER_W // _BLK
_GPB = _BLK // _LANES


def _lookup_body(de_hbm, w_hbm, b_hbm, feat_hbm, out_hbm,
                 tbl_v, de_v, w_v, b_v, feat_v, rows_v, sem_f, sem_out):
    c = lax.axis_index("c")
    s = lax.axis_index("s")
    wid = c * _NS + s
    ebase = wid * _PER_W

    pltpu.async_copy(feat_hbm.at[pl.ds(ebase, _BLK)], feat_v.at[pl.ds(0, _BLK)], sem_f)

    pltpu.sync_copy(de_hbm, de_v)
    pltpu.sync_copy(w_hbm, w_v)
    pltpu.sync_copy(b_hbm, b_v)

    def trow(r, carry):
        def kchunk(kc, acc):
            dv = de_v[pl.ds(r * _D + kc * _LANES, _LANES)]
            acc = list(acc)
            for l in range(_LANES):
                a = dv[l]
                kk = kc * _LANES + l
                for m in range(_MV):
                    acc[m] = acc[m] + a * w_v[pl.ds(kk * _D + m * _LANES, _LANES)]
            return tuple(acc)

        acc = lax.fori_loop(
            0, _D // _LANES, kchunk,
            tuple(b_v[pl.ds(m * _LANES, _LANES)] for m in range(_MV)),
        )
        for m in range(_MV):
            tbl_v[pl.ds(r * _D + m * _LANES, _LANES)] = acc[m]
        return carry

    lax.fori_loop(1, _CUT, trow, 0)

    def block(j, carry):
        b = j % 2
        foff = b * _BLK
        roff = b * _BLK * _D
        pltpu.make_async_copy(
            feat_hbm.at[pl.ds(ebase, _BLK)], feat_v.at[pl.ds(0, _BLK)], sem_f
        ).wait()

        @pl.when(j + 1 < _NB)
        def _():
            pltpu.async_copy(
                feat_hbm.at[pl.ds(ebase + (j + 1) * _BLK, _BLK)],
                feat_v.at[pl.ds((1 - b) * _BLK, _BLK)],
                sem_f,
            )

        @pl.when(j >= 2)
        def _():
            pltpu.make_async_copy(
                rows_v.at[pl.ds(roff, _BLK * _D)],
                out_hbm.at[pl.ds((ebase + (j - 2) * _BLK) * _D, _BLK * _D)],
                sem_out,
            ).wait()

        @plsc.parallel_loop(0, _GPB, 1)
        def _group(g):
            x = feat_v[pl.ds(foff + g * _LANES, _LANES)]
            xiv = jnp.clip(x, 1.0, _CUT - 1e-6).astype(jnp.int32) * _D
            ro0 = roff + g * (_LANES * _D)
            for l in range(_LANES):
                si = xiv[l]
                ro = ro0 + l * _D
                for c0 in range(0, _D, _LANES):
                    rows_v[pl.ds(ro + c0, _LANES)] = tbl_v[pl.ds(si + c0, _LANES)]
        pltpu.async_copy(
            rows_v.at[pl.ds(roff, _BLK * _D)],
            out_hbm.at[pl.ds((ebase + j * _BLK) * _D, _BLK * _D)],
            sem_out,
        )
        return carry

    lax.fori_loop(0, _NB, block, 0)
    for j in (_NB - 2, _NB - 1):
        roff = (j % 2) * _BLK * _D
        pltpu.make_async_copy(
            rows_v.at[pl.ds(roff, _BLK * _D)],
            out_hbm.at[pl.ds((ebase + j * _BLK) * _D, _BLK * _D)],
            sem_out,
        ).wait()


def kernel(dist_feat, dist_embed, W_in, b_in):
    mesh = plsc.VectorSubcoreMesh(core_axis_name="c", subcore_axis_name="s")
    lookup = pl.kernel(
        _lookup_body,
        out_type=jax.ShapeDtypeStruct((_E * _D,), jnp.float32),
        mesh=mesh,
        compiler_params=pltpu.CompilerParams(needs_layout_passes=False),
        scratch_types=[
            pltpu.VMEM(((_CUT + 1) * _D,), jnp.float32),
            pltpu.VMEM(((_CUT + 1) * _D,), jnp.float32),
            pltpu.VMEM((_D * _D,), jnp.float32),
            pltpu.VMEM((_D,), jnp.float32),
            pltpu.VMEM((2 * _BLK,), jnp.float32),
            pltpu.VMEM((2 * _BLK * _D,), jnp.float32),
            pltpu.SemaphoreType.DMA,
            pltpu.SemaphoreType.DMA,
        ],
    )
    out = lookup(
        dist_embed.reshape((_CUT + 1) * _D),
        W_in.reshape(_D * _D),
        b_in,
        dist_feat,
    )
    return out.reshape(_E, _D)


# speedup vs baseline: 1.0343x; 1.0343x over previous
import jax
import jax.numpy as jnp
from jax import lax
from jax.experimental import pallas as pl
from jax.experimental.pallas import tpu as pltpu
from jax.experimental.pallas import tpu_sc as plsc

_CUT = 5
_E = 320000
_D = 128

_NC = 2
_NS = 16
_NW = _NC * _NS
_LANES = 16
_MV = _D // _LANES

_PER_W = _E // _NW
_BLK = 400
_NB = _PER_W // _BLK
_GPB = _BLK // _LANES


def _lookup_body(de_hbm, w_hbm, b_hbm, feat_hbm, out_hbm,
                 tbl_sh, tbl_v, de_v, w_v, b_v, feat_v, rows_v, sem_f, sem_out):
    c = lax.axis_index("c")
    s = lax.axis_index("s")
    wid = c * _NS + s
    ebase = wid * _PER_W

    pltpu.async_copy(feat_hbm.at[pl.ds(ebase, _BLK)], feat_v.at[pl.ds(0, _BLK)], sem_f)

    pltpu.sync_copy(de_hbm, de_v)
    pltpu.sync_copy(w_hbm, w_v)
    pltpu.sync_copy(b_hbm, b_v)

    for qi in range(2):
        q = s * 2 + qi
        r = 1 + q // _MV
        m = q % _MV

        def kchunk(kc, acc, r=r, m=m):
            dv = de_v[pl.ds(r * _D + kc * _LANES, _LANES)]
            for l in range(_LANES):
                acc = acc + dv[l] * w_v[pl.ds((kc * _LANES + l) * _D + m * _LANES, _LANES)]
            return acc

        acc = lax.fori_loop(0, _D // _LANES, kchunk, b_v[pl.ds(m * _LANES, _LANES)])
        tbl_v[pl.ds(qi * _LANES, _LANES)] = acc
        pltpu.sync_copy(tbl_v.at[pl.ds(qi * _LANES, _LANES)],
                        tbl_sh.at[pl.ds(q * _LANES, _LANES)])
    plsc.subcore_barrier()
    pltpu.sync_copy(tbl_sh, tbl_v.at[pl.ds(_D, 4 * _D)])

    def block(j, carry):
        b = j % 2
        foff = b * _BLK
        roff = b * _BLK * _D
        pltpu.make_async_copy(
            feat_hbm.at[pl.ds(ebase, _BLK)], feat_v.at[pl.ds(0, _BLK)], sem_f
        ).wait()

        @pl.when(j + 1 < _NB)
        def _():
            pltpu.async_copy(
                feat_hbm.at[pl.ds(ebase + (j + 1) * _BLK, _BLK)],
                feat_v.at[pl.ds((1 - b) * _BLK, _BLK)],
                sem_f,
            )

        @pl.when(j >= 2)
        def _():
            pltpu.make_async_copy(
                rows_v.at[pl.ds(roff, _BLK * _D)],
                out_hbm.at[pl.ds((ebase + (j - 2) * _BLK) * _D, _BLK * _D)],
                sem_out,
            ).wait()

        @plsc.parallel_loop(0, _GPB, 1)
        def _group(g):
            x = feat_v[pl.ds(foff + g * _LANES, _LANES)]
            xiv = jnp.clip(x, 1.0, _CUT - 1e-6).astype(jnp.int32) * _D
            ro0 = roff + g * (_LANES * _D)
            for l in range(_LANES):
                si = xiv[l]
                ro = ro0 + l * _D
                for c0 in range(0, _D, _LANES):
                    rows_v[pl.ds(ro + c0, _LANES)] = tbl_v[pl.ds(si + c0, _LANES)]
        pltpu.async_copy(
            rows_v.at[pl.ds(roff, _BLK * _D)],
            out_hbm.at[pl.ds((ebase + j * _BLK) * _D, _BLK * _D)],
            sem_out,
        )
        return carry

    lax.fori_loop(0, _NB, block, 0)
    for j in (_NB - 2, _NB - 1):
        roff = (j % 2) * _BLK * _D
        pltpu.make_async_copy(
            rows_v.at[pl.ds(roff, _BLK * _D)],
            out_hbm.at[pl.ds((ebase + j * _BLK) * _D, _BLK * _D)],
            sem_out,
        ).wait()


def kernel(dist_feat, dist_embed, W_in, b_in):
    mesh = plsc.VectorSubcoreMesh(core_axis_name="c", subcore_axis_name="s")
    lookup = pl.kernel(
        _lookup_body,
        out_type=jax.ShapeDtypeStruct((_E * _D,), jnp.float32),
        mesh=mesh,
        compiler_params=pltpu.CompilerParams(needs_layout_passes=False),
        scratch_types=[
            pltpu.VMEM_SHARED((4 * _D,), jnp.float32),
            pltpu.VMEM(((_CUT + 1) * _D,), jnp.float32),
            pltpu.VMEM(((_CUT + 1) * _D,), jnp.float32),
            pltpu.VMEM((_D * _D,), jnp.float32),
            pltpu.VMEM((_D,), jnp.float32),
            pltpu.VMEM((2 * _BLK,), jnp.float32),
            pltpu.VMEM((2 * _BLK * _D,), jnp.float32),
            pltpu.SemaphoreType.DMA,
            pltpu.SemaphoreType.DMA,
        ],
    )
    out = lookup(
        dist_embed.reshape((_CUT + 1) * _D),
        W_in.reshape(_D * _D),
        b_in,
        dist_feat,
    )
    return out.reshape(_E, _D)
